# Initial kernel scaffold; baseline (speedup 1.0000x reference)
import jax
import jax.numpy as jnp
from jax.experimental import pallas as pl


def kernel(u_idx, r_idx, v_idx, i_to_corrupt, Eh, rvh_w, Wh, bs, bo):
    raise NotImplementedError("write your pallas kernel here")


# speedup vs baseline: 2.0615x; 2.0615x over previous
import functools

import jax
import jax.numpy as jnp
from jax import lax
from jax.experimental import pallas as pl
from jax.experimental.pallas import tpu as pltpu
from jax.experimental.pallas import tpu_sc as plsc

NUM_ENT = 100000
NUM_REL = 1000
DIM = 128
B = 4096
EPS = 1e-5

_NC = 2
_NS = 16
_NW = _NC * _NS
_BPW = B // _NW

_F32 = jnp.float32


def _sc_gather(u_idx, r_idx, v_idx, Eh, rvh_w, Wh, bs, bo):
    mesh = plsc.VectorSubcoreMesh(core_axis_name="c", subcore_axis_name="s")
    out_type = (
        jax.ShapeDtypeStruct((B, DIM), _F32),
        jax.ShapeDtypeStruct((B, DIM), _F32),
        jax.ShapeDtypeStruct((B, DIM), _F32),
        jax.ShapeDtypeStruct((B, DIM), _F32),
        jax.ShapeDtypeStruct((B,), _F32),
        jax.ShapeDtypeStruct((B,), _F32),
    )

    @functools.partial(
        pl.kernel,
        mesh=mesh,
        out_type=out_type,
        scratch_types=[
            pltpu.VMEM((_BPW,), jnp.int32),
            pltpu.VMEM((_BPW,), jnp.int32),
            pltpu.VMEM((_BPW,), jnp.int32),
            pltpu.VMEM((_BPW, DIM), _F32),
            pltpu.VMEM((_BPW, DIM), _F32),
            pltpu.VMEM((_BPW, DIM), _F32),
            pltpu.VMEM((_BPW, DIM), _F32),
            pltpu.VMEM((_BPW,), _F32),
            pltpu.VMEM((_BPW,), _F32),
            pltpu.SemaphoreType.DMA,
            pltpu.SemaphoreType.DMA,
            pltpu.SemaphoreType.DMA,
            pltpu.SemaphoreType.DMA,
            pltpu.SemaphoreType.DMA,
            pltpu.SemaphoreType.DMA,
        ],
    )
    def k(u_idx_h, r_idx_h, v_idx_h, eh_h, rvh_h, wh_h, bs_h, bo_h,
          u_o, v_o, r_o, w_o, bsu_o, bov_o,
          uix, rix, vix, ub, vb, rb, wb, bsb, bob,
          s0, s1, s2, s3, s4, s5):
        wid = lax.axis_index("s") * _NC + lax.axis_index("c")
        base = wid * _BPW
        pltpu.sync_copy(u_idx_h.at[pl.ds(base, _BPW)], uix)
        pltpu.sync_copy(v_idx_h.at[pl.ds(base, _BPW)], vix)
        pltpu.sync_copy(r_idx_h.at[pl.ds(base, _BPW)], rix)
        c0 = pltpu.async_copy(eh_h.at[uix], ub, s0)
        c1 = pltpu.async_copy(eh_h.at[vix], vb, s1)
        c2 = pltpu.async_copy(rvh_h.at[rix], rb, s2)
        c3 = pltpu.async_copy(wh_h.at[rix], wb, s3)
        c4 = pltpu.async_copy(bs_h.at[uix], bsb, s4)
        c5 = pltpu.async_copy(bo_h.at[vix], bob, s5)
        c0.wait()
        c1.wait()
        c2.wait()
        c3.wait()
        c4.wait()
        c5.wait()
        pltpu.sync_copy(ub, u_o.at[pl.ds(base, _BPW)])
        pltpu.sync_copy(vb, v_o.at[pl.ds(base, _BPW)])
        pltpu.sync_copy(rb, r_o.at[pl.ds(base, _BPW)])
        pltpu.sync_copy(wb, w_o.at[pl.ds(base, _BPW)])
        pltpu.sync_copy(bsb, bsu_o.at[pl.ds(base, _BPW)])
        pltpu.sync_copy(bob, bov_o.at[pl.ds(base, _BPW)])

    return k(u_idx, r_idx, v_idx, Eh, rvh_w, Wh, bs, bo)


def _math_body(u_ref, v_ref, r_ref, w_ref, bsu_ref, bov_ref, o_ref):
    u = u_ref[...]
    v = v_ref[...]
    rv = r_ref[...]
    w = w_ref[...]

    def norm1(x):
        n = jnp.sqrt(jnp.sum(x * x, axis=-1, keepdims=True))
        scale = jnp.where(n >= 1.0, (1.0 - EPS) / jnp.maximum(n, 1e-10), 1.0)
        return x * scale

    def atanh(n):
        return 0.5 * jnp.log((1.0 + n) / (1.0 - n))

    def log_map(x):
        nx = jnp.clip(jnp.sqrt(jnp.sum(x * x, axis=-1, keepdims=True)),
                      1e-10, 1.0 - 1e-5)
        return atanh(nx) * x / nx

    def exp_map(x):
        nx = jnp.clip(jnp.sqrt(jnp.sum(x * x, axis=-1, keepdims=True)),
                      1e-10, None)
        return jnp.tanh(nx) * x / nx

    def mobius_add(x, y):
        sqx = jnp.clip(jnp.sum(x * x, axis=-1, keepdims=True), 0.0, 1.0 - 1e-5)
        sqy = jnp.clip(jnp.sum(y * y, axis=-1, keepdims=True), 0.0, 1.0 - 1e-5)
        dot = jnp.sum(x * y, axis=-1, keepdims=True)
        num = (1.0 + 2.0 * dot + sqy) * x + (1.0 - sqx) * y
        den = 1.0 + 2.0 * dot + sqx * sqy
        return num / den

    un = norm1(u)
    vn = norm1(v)
    rvn = norm1(rv)
    head = norm1(exp_map(w * log_map(un)))
    tail = norm1(mobius_add(vn, rvn))
    m = mobius_add(-head, tail)
    nm = jnp.clip(jnp.sqrt(jnp.sum(m * m, axis=-1)), 1e-10, 1.0 - 1e-5)
    dist = (2.0 * atanh(nm)) ** 2
    o_ref[0, 0, :] = -dist + bsu_ref[0, 0, :] + bov_ref[0, 0, :]


_BLK = 512
_G = B // _BLK


def _tc_math(u_rows, v_rows, r_rows, w_rows, bs_u, bo_v):
    row_spec = pl.BlockSpec((_BLK, DIM), lambda g: (g, 0))
    sc_spec = pl.BlockSpec((1, 1, _BLK), lambda g: (g, 0, 0))
    out = pl.pallas_call(
        _math_body,
        grid=(_G,),
        in_specs=[row_spec, row_spec, row_spec, row_spec, sc_spec, sc_spec],
        out_specs=sc_spec,
        out_shape=jax.ShapeDtypeStruct((_G, 1, _BLK), _F32),
    )(u_rows, v_rows, r_rows, w_rows,
      bs_u.reshape(_G, 1, _BLK), bo_v.reshape(_G, 1, _BLK))
    return out.reshape(B)


def kernel(u_idx, r_idx, v_idx, i_to_corrupt, Eh, rvh_w, Wh, bs, bo):
    del i_to_corrupt
    u_idx = u_idx.astype(jnp.int32)
    r_idx = r_idx.astype(jnp.int32)
    v_idx = v_idx.astype(jnp.int32)
    u_rows, v_rows, r_rows, w_rows, bs_u, bo_v = _sc_gather(
        u_idx, r_idx, v_idx, Eh, rvh_w, Wh, bs, bo)
    return _tc_math(u_rows, v_rows, r_rows, w_rows, bs_u, bo_v)
